# baseline (speedup 1.0000x reference)
import functools

import jax
import jax.numpy as jnp
from jax.experimental import pallas as pl
from jax.experimental.pallas import tpu as pltpu

_LN_EPS = 1e-5


def _ffn_body(x_ref, w1_ref, b1_ref, w2_ref, b2_ref, g_ref, bt_ref, o_ref,
              *, inv_d):
    s_blk = x_ref.shape[0]
    w1 = w1_ref[...].astype(jnp.bfloat16)
    w2 = w2_ref[...].astype(jnp.bfloat16)
    b1 = b1_ref[...]
    b2 = b2_ref[...]
    g = g_ref[...]
    bt = bt_ref[...]

    for s in range(s_blk):
        x = x_ref[s]
        h = jnp.dot(w1, x.astype(jnp.bfloat16),
                    preferred_element_type=jnp.float32)
        h = jnp.maximum((h + b1).astype(jnp.bfloat16), 0)
        y = jnp.dot(w2, h, preferred_element_type=jnp.float32)
        z = y + b2 + x
        mean = jnp.sum(z, axis=0, keepdims=True) * inv_d
        d_c = z - mean
        var = jnp.sum(d_c * d_c, axis=0, keepdims=True) * inv_d
        zn = d_c * jax.lax.rsqrt(var + _LN_EPS)
        o_ref[s] = (zn * g + bt).astype(o_ref.dtype)


def kernel(x, w1t, b1, w2t, b2, gamma, beta):
    seq, batch, d = x.shape
    f = w1t.shape[1]
    dtype = x.dtype

    xt = jnp.transpose(x, (0, 2, 1))
    w1 = w1t.T
    w2 = w2t.T
    b1r = b1.reshape(f, 1)
    b2r = b2.reshape(d, 1)
    gr = gamma.reshape(d, 1)
    btr = beta.reshape(d, 1)

    s_blk = min(8, seq)
    b_blk = 256 if batch % 256 == 0 else batch
    grid = (pl.cdiv(seq, s_blk), pl.cdiv(batch, b_blk))

    n = seq * batch
    flops = 2 * n * d * f * 2 + 8 * n * d
    bytes_accessed = 4 * (2 * n * d + 2 * d * f + f + 3 * d)
    cost = pl.CostEstimate(flops=int(flops), transcendentals=int(n),
                           bytes_accessed=int(bytes_accessed))

    out_t = pl.pallas_call(
        functools.partial(_ffn_body, inv_d=1.0 / d),
        out_shape=jax.ShapeDtypeStruct((seq, d, batch), dtype),
        grid_spec=pltpu.PrefetchScalarGridSpec(
            num_scalar_prefetch=0,
            grid=grid,
            in_specs=[
                pl.BlockSpec((s_blk, d, b_blk), lambda i, j: (i, 0, j)),
                pl.BlockSpec((f, d), lambda i, j: (0, 0)),
                pl.BlockSpec((f, 1), lambda i, j: (0, 0)),
                pl.BlockSpec((d, f), lambda i, j: (0, 0)),
                pl.BlockSpec((d, 1), lambda i, j: (0, 0)),
                pl.BlockSpec((d, 1), lambda i, j: (0, 0)),
                pl.BlockSpec((d, 1), lambda i, j: (0, 0)),
            ],
            out_specs=pl.BlockSpec((s_blk, d, b_blk),
                                   lambda i, j: (i, 0, j)),
        ),
        compiler_params=pltpu.CompilerParams(
            dimension_semantics=("parallel", "arbitrary")),
        cost_estimate=cost,
    )(xt, w1, b1r, w2, b2r, gr, btr)

    return jnp.transpose(out_t, (0, 2, 1))


# speedup vs baseline: 1.0968x; 1.0968x over previous
import functools

import jax
import jax.numpy as jnp
from jax.experimental import pallas as pl
from jax.experimental.pallas import tpu as pltpu

_LN_EPS = 1e-5


def _ffn_body(x_ref, w1_ref, b1_ref, w2_ref, b2_ref, g_ref, bt_ref, o_ref,
              *, inv_d):
    s_blk = x_ref.shape[0]
    w1 = w1_ref[...].astype(jnp.bfloat16)
    w2 = w2_ref[...].astype(jnp.bfloat16)
    b1 = b1_ref[...]
    b2 = b2_ref[...]
    g = g_ref[...]
    bt = bt_ref[...]

    batch = x_ref.shape[2]
    b_blk = 256 if batch % 256 == 0 else batch
    for s in range(s_blk):
        for c in range(batch // b_blk):
            sl = pl.ds(c * b_blk, b_blk)
            x = x_ref[s, :, sl]
            h = jnp.dot(w1, x.astype(jnp.bfloat16),
                        preferred_element_type=jnp.float32)
            h = jnp.maximum((h + b1).astype(jnp.bfloat16), 0)
            y = jnp.dot(w2, h, preferred_element_type=jnp.float32)
            z = y + b2 + x
            mean = jnp.sum(z, axis=0, keepdims=True) * inv_d
            d_c = z - mean
            var = jnp.sum(d_c * d_c, axis=0, keepdims=True) * inv_d
            zn = d_c * jax.lax.rsqrt(var + _LN_EPS)
            o_ref[s, :, sl] = (zn * g + bt).astype(o_ref.dtype)


def kernel(x, w1t, b1, w2t, b2, gamma, beta):
    seq, batch, d = x.shape
    f = w1t.shape[1]
    dtype = x.dtype

    xt = jnp.transpose(x, (0, 2, 1))
    w1 = w1t.T
    w2 = w2t.T
    b1r = b1.reshape(f, 1)
    b2r = b2.reshape(d, 1)
    gr = gamma.reshape(d, 1)
    btr = beta.reshape(d, 1)

    s_blk = min(8, seq)
    grid = (pl.cdiv(seq, s_blk),)

    n = seq * batch
    flops = 2 * n * d * f * 2 + 8 * n * d
    bytes_accessed = 4 * (2 * n * d + 2 * d * f + f + 3 * d)
    cost = pl.CostEstimate(flops=int(flops), transcendentals=int(n),
                           bytes_accessed=int(bytes_accessed))

    out_t = pl.pallas_call(
        functools.partial(_ffn_body, inv_d=1.0 / d),
        out_shape=jax.ShapeDtypeStruct((seq, d, batch), dtype),
        grid_spec=pltpu.PrefetchScalarGridSpec(
            num_scalar_prefetch=0,
            grid=grid,
            in_specs=[
                pl.BlockSpec((s_blk, d, batch), lambda i: (i, 0, 0)),
                pl.BlockSpec((f, d), lambda i: (0, 0)),
                pl.BlockSpec((f, 1), lambda i: (0, 0)),
                pl.BlockSpec((d, f), lambda i: (0, 0)),
                pl.BlockSpec((d, 1), lambda i: (0, 0)),
                pl.BlockSpec((d, 1), lambda i: (0, 0)),
                pl.BlockSpec((d, 1), lambda i: (0, 0)),
            ],
            out_specs=pl.BlockSpec((s_blk, d, batch), lambda i: (i, 0, 0)),
        ),
        compiler_params=pltpu.CompilerParams(
            dimension_semantics=("parallel",)),
        cost_estimate=cost,
    )(xt, w1, b1r, w2, b2r, gr, btr)

    return jnp.transpose(out_t, (0, 2, 1))


# speedup vs baseline: 2.6222x; 2.3908x over previous
import functools

import jax
import jax.numpy as jnp
from jax.experimental import pallas as pl
from jax.experimental.pallas import tpu as pltpu

_LN_EPS = 1e-5

_W2_ROW = 136
_PROWS = 176
_PCOLS = 256


def _ffn_body(x_ref, p_ref, o_ref, *, inv_d, d, f):
    s_blk, _, batch = x_ref.shape
    w1a = p_ref[0:136, 0:d + 1].astype(jnp.bfloat16)
    w2a = p_ref[_W2_ROW:_PROWS, 0:f].astype(jnp.bfloat16)
    b2c = p_ref[_W2_ROW:_W2_ROW + d, 128:129]
    g = p_ref[_W2_ROW:_W2_ROW + d, 129:130]
    bt = p_ref[_W2_ROW:_W2_ROW + d, 130:131]
    ones_row = jnp.ones((1, batch), jnp.bfloat16)
    ones_d = jnp.ones((1, d), jnp.bfloat16)

    for s in range(s_blk):
        x = x_ref[s]
        xaug = jnp.concatenate([x.astype(jnp.bfloat16), ones_row], axis=0)
        ha = jnp.dot(w1a, xaug, preferred_element_type=jnp.float32)
        hr = jnp.maximum(ha[0:f].astype(jnp.bfloat16), 0)
        sx = ha[f:f + 1]
        ya = jnp.dot(w2a, hr, preferred_element_type=jnp.float32)
        y = ya[0:d]
        sy = ya[d:d + 1]
        mean = (sx + sy) * inv_d
        z = (y + x) + b2c
        d_c = z - mean
        var = jnp.sum(d_c * d_c, axis=0, keepdims=True) * inv_d
        rs = jax.lax.rsqrt(var + _LN_EPS)
        o_ref[s] = (d_c * rs * g + bt).astype(o_ref.dtype)


def _build_params(w1t, b1, w2t, b2, gamma, beta, d, f):
    w1 = w1t.T
    top = jnp.concatenate([w1, b1.reshape(f, 1)], axis=1)
    bot = jnp.concatenate([jnp.ones((1, d), jnp.float32),
                           jnp.sum(b2).reshape(1, 1)], axis=1)
    w1aug = jnp.concatenate([top, bot], axis=0)

    w2 = w2t.T
    w2aug = jnp.concatenate([w2, jnp.sum(w2, axis=0, keepdims=True)], axis=0)

    p = jnp.zeros((_PROWS, _PCOLS), jnp.float32)
    p = jax.lax.dynamic_update_slice(p, w1aug, (0, 0))
    p = jax.lax.dynamic_update_slice(p, w2aug, (_W2_ROW, 0))
    vecs = jnp.stack([b2, gamma, beta], axis=1)
    p = jax.lax.dynamic_update_slice(p, vecs, (_W2_ROW, 128))
    return p


def kernel(x, w1t, b1, w2t, b2, gamma, beta):
    seq, batch, d = x.shape
    f = w1t.shape[1]
    dtype = x.dtype

    xt = jnp.transpose(x, (0, 2, 1))
    params = _build_params(w1t, b1, w2t, b2, gamma, beta, d, f)

    s_blk = min(8, seq)
    grid = (pl.cdiv(seq, s_blk),)

    n = seq * batch
    flops = 2 * n * d * f * 2 + 8 * n * d
    bytes_accessed = 4 * (2 * n * d + _PROWS * _PCOLS)
    cost = pl.CostEstimate(flops=int(flops), transcendentals=int(n),
                           bytes_accessed=int(bytes_accessed))

    out_t = pl.pallas_call(
        functools.partial(_ffn_body, inv_d=1.0 / d, d=d, f=f),
        out_shape=jax.ShapeDtypeStruct((seq, d, batch), dtype),
        grid_spec=pltpu.PrefetchScalarGridSpec(
            num_scalar_prefetch=0,
            grid=grid,
            in_specs=[
                pl.BlockSpec((s_blk, d, batch), lambda i: (i, 0, 0)),
                pl.BlockSpec((_PROWS, _PCOLS), lambda i: (0, 0)),
            ],
            out_specs=pl.BlockSpec((s_blk, d, batch), lambda i: (i, 0, 0)),
        ),
        compiler_params=pltpu.CompilerParams(
            dimension_semantics=("parallel",)),
        cost_estimate=cost,
    )(xt, params)

    return jnp.transpose(out_t, (0, 2, 1))


# speedup vs baseline: 2.6808x; 1.0224x over previous
import functools

import jax
import jax.numpy as jnp
from jax.experimental import pallas as pl
from jax.experimental.pallas import tpu as pltpu

_LN_EPS = 1e-5

_W2_ROW = 136
_PROWS = 176
_PCOLS = 256


def _ffn_body(x_ref, p_ref, o_ref, *, inv_d, d, f):
    s_blk, _, batch = x_ref.shape
    w1a = p_ref[0:136, 0:d + 1].astype(jnp.bfloat16)
    w2a = p_ref[_W2_ROW:_PROWS, 0:f].astype(jnp.bfloat16)
    b2c = p_ref[_W2_ROW:_W2_ROW + d, 128:129]
    g = p_ref[_W2_ROW:_W2_ROW + d, 129:130]
    bt = p_ref[_W2_ROW:_W2_ROW + d, 130:131]
    ones_row = jnp.ones((1, batch), jnp.bfloat16)
    ones_d = jnp.ones((1, d), jnp.bfloat16)

    for s in range(s_blk):
        x = x_ref[s]
        xaug = jnp.concatenate([x.astype(jnp.bfloat16), ones_row], axis=0)
        ha = jnp.dot(w1a, xaug, preferred_element_type=jnp.float32)
        hr = jnp.maximum(ha[0:f].astype(jnp.bfloat16), 0)
        sx = ha[f:f + 1]
        ya = jnp.dot(w2a, hr, preferred_element_type=jnp.float32)
        y = ya[0:d]
        sy = ya[d:d + 1]
        mean = (sx + sy) * inv_d
        z = (y + x) + b2c
        d_c = z - mean
        var = jnp.sum(d_c * d_c, axis=0, keepdims=True) * inv_d
        rs = jax.lax.rsqrt(var + _LN_EPS)
        o_ref[s] = (d_c * rs * g + bt).astype(o_ref.dtype)


def _build_params(w1t, b1, w2t, b2, gamma, beta, d, f):
    w1 = w1t.T
    top = jnp.concatenate([w1, b1.reshape(f, 1)], axis=1)
    bot = jnp.concatenate([jnp.ones((1, d), jnp.float32),
                           jnp.sum(b2).reshape(1, 1)], axis=1)
    w1aug = jnp.concatenate([top, bot], axis=0)

    w2 = w2t.T
    w2aug = jnp.concatenate([w2, jnp.sum(w2, axis=0, keepdims=True)], axis=0)

    p = jnp.zeros((_PROWS, _PCOLS), jnp.float32)
    p = jax.lax.dynamic_update_slice(p, w1aug, (0, 0))
    p = jax.lax.dynamic_update_slice(p, w2aug, (_W2_ROW, 0))
    vecs = jnp.stack([b2, gamma, beta], axis=1)
    p = jax.lax.dynamic_update_slice(p, vecs, (_W2_ROW, 128))
    return p


def kernel(x, w1t, b1, w2t, b2, gamma, beta):
    seq, batch, d = x.shape
    f = w1t.shape[1]
    dtype = x.dtype

    xt = jnp.transpose(x, (0, 2, 1))
    params = _build_params(w1t, b1, w2t, b2, gamma, beta, d, f)

    s_blk = min(16, seq)
    grid = (pl.cdiv(seq, s_blk),)

    n = seq * batch
    flops = 2 * n * d * f * 2 + 8 * n * d
    bytes_accessed = 4 * (2 * n * d + _PROWS * _PCOLS)
    cost = pl.CostEstimate(flops=int(flops), transcendentals=int(n),
                           bytes_accessed=int(bytes_accessed))

    out_t = pl.pallas_call(
        functools.partial(_ffn_body, inv_d=1.0 / d, d=d, f=f),
        out_shape=jax.ShapeDtypeStruct((seq, d, batch), dtype),
        grid_spec=pltpu.PrefetchScalarGridSpec(
            num_scalar_prefetch=0,
            grid=grid,
            in_specs=[
                pl.BlockSpec((s_blk, d, batch), lambda i: (i, 0, 0)),
                pl.BlockSpec((_PROWS, _PCOLS), lambda i: (0, 0)),
            ],
            out_specs=pl.BlockSpec((s_blk, d, batch), lambda i: (i, 0, 0)),
        ),
        compiler_params=pltpu.CompilerParams(
            dimension_semantics=("parallel",)),
        cost_estimate=cost,
    )(xt, params)

    return jnp.transpose(out_t, (0, 2, 1))


# speedup vs baseline: 2.8813x; 1.0748x over previous
import functools

import jax
import jax.numpy as jnp
from jax.experimental import pallas as pl
from jax.experimental.pallas import tpu as pltpu

_LN_EPS = 1e-5


def _ffn_body(x_ref, w1_ref, b1_ref, w2_ref, b2_ref, g_ref, bt_ref, o_ref,
              *, inv_d):
    s_blk, d, batch = x_ref.shape
    f = w1_ref.shape[0]

    w1 = w1_ref[...]
    b1c = jnp.transpose(b1_ref[...])
    sb2 = jnp.sum(b2_ref[...], axis=1, keepdims=True)
    row = jnp.concatenate([jnp.ones((1, d), jnp.float32), sb2], axis=1)
    w1a = jnp.concatenate(
        [jnp.concatenate([w1, b1c], axis=1), row], axis=0
    ).astype(jnp.bfloat16)

    w2 = w2_ref[...]
    w2a = jnp.concatenate(
        [w2, jnp.sum(w2, axis=0, keepdims=True)], axis=0
    ).astype(jnp.bfloat16)

    b2c = jnp.transpose(b2_ref[...])
    g = jnp.transpose(g_ref[...])
    bt = jnp.transpose(bt_ref[...])
    ones_row = jnp.ones((1, batch), jnp.bfloat16)

    for s in range(s_blk):
        x = x_ref[s]
        xaug = jnp.concatenate([x.astype(jnp.bfloat16), ones_row], axis=0)
        ha = jnp.dot(w1a, xaug, preferred_element_type=jnp.float32)
        hr = jnp.maximum(ha[0:f].astype(jnp.bfloat16), 0)
        sx = ha[f:f + 1]
        ya = jnp.dot(w2a, hr, preferred_element_type=jnp.float32)
        y = ya[0:d]
        sy = ya[d:d + 1]
        mean = (sx + sy) * inv_d
        z = (y + x) + b2c
        d_c = z - mean
        var = jnp.sum(d_c * d_c, axis=0, keepdims=True) * inv_d
        rs = jax.lax.rsqrt(var + _LN_EPS)
        o_ref[s] = (d_c * rs * g + bt).astype(o_ref.dtype)


def kernel(x, w1t, b1, w2t, b2, gamma, beta):
    seq, batch, d = x.shape
    f = w1t.shape[1]
    dtype = x.dtype

    xt = jnp.transpose(x, (0, 2, 1))
    w1 = w1t.T
    w2 = w2t.T
    b1r = b1.reshape(1, f)
    b2r = b2.reshape(1, d)
    gr = gamma.reshape(1, d)
    btr = beta.reshape(1, d)

    s_blk = min(32, seq)
    grid = (pl.cdiv(seq, s_blk),)

    n = seq * batch
    flops = 2 * n * d * f * 2 + 8 * n * d
    bytes_accessed = 4 * (2 * n * d + 2 * d * f + f + 3 * d)
    cost = pl.CostEstimate(flops=int(flops), transcendentals=int(n),
                           bytes_accessed=int(bytes_accessed))

    out_t = pl.pallas_call(
        functools.partial(_ffn_body, inv_d=1.0 / d),
        out_shape=jax.ShapeDtypeStruct((seq, d, batch), dtype),
        grid_spec=pltpu.PrefetchScalarGridSpec(
            num_scalar_prefetch=0,
            grid=grid,
            in_specs=[
                pl.BlockSpec((s_blk, d, batch), lambda i: (i, 0, 0)),
                pl.BlockSpec((f, d), lambda i: (0, 0)),
                pl.BlockSpec((1, f), lambda i: (0, 0)),
                pl.BlockSpec((d, f), lambda i: (0, 0)),
                pl.BlockSpec((1, d), lambda i: (0, 0)),
                pl.BlockSpec((1, d), lambda i: (0, 0)),
                pl.BlockSpec((1, d), lambda i: (0, 0)),
            ],
            out_specs=pl.BlockSpec((s_blk, d, batch), lambda i: (i, 0, 0)),
        ),
        compiler_params=pltpu.CompilerParams(
            dimension_semantics=("parallel",)),
        cost_estimate=cost,
    )(xt, w1, b1r, w2, b2r, gr, btr)

    return jnp.transpose(out_t, (0, 2, 1))


# speedup vs baseline: 2.9034x; 1.0077x over previous
import functools

import jax
import jax.numpy as jnp
from jax.experimental import pallas as pl
from jax.experimental.pallas import tpu as pltpu

_LN_EPS = 1e-5


def _ffn_body(x_ref, w1_ref, b1_ref, w2_ref, b2_ref, g_ref, bt_ref, o_ref,
              *, inv_d):
    s_blk, d, batch = x_ref.shape
    f = w1_ref.shape[0]

    w1 = w1_ref[...]
    b1c = jnp.transpose(b1_ref[...]).astype(jnp.bfloat16)
    sb2 = jnp.sum(b2_ref[...], axis=1, keepdims=True)
    w1a = jnp.concatenate(
        [w1, jnp.ones((1, d), jnp.float32)], axis=0
    ).astype(jnp.bfloat16)

    w2 = w2_ref[...]
    w2a = jnp.concatenate(
        [w2, jnp.sum(w2, axis=0, keepdims=True)], axis=0
    ).astype(jnp.bfloat16)

    b2c = jnp.transpose(b2_ref[...])
    g = jnp.transpose(g_ref[...])
    bt = jnp.transpose(bt_ref[...])

    for s in range(s_blk):
        x = x_ref[s]
        ha = jnp.dot(w1a, x.astype(jnp.bfloat16),
                     preferred_element_type=jnp.float32)
        hr = jnp.maximum(ha[0:f].astype(jnp.bfloat16) + b1c, 0)
        sx = ha[f:f + 1]
        ya = jnp.dot(w2a, hr, preferred_element_type=jnp.float32)
        y = ya[0:d]
        sy = ya[d:d + 1]
        mean = (sx + sy + sb2) * inv_d
        z = (y + x) + b2c
        d_c = z - mean
        var = jnp.sum(d_c * d_c, axis=0, keepdims=True) * inv_d
        rs = jax.lax.rsqrt(var + _LN_EPS)
        o_ref[s] = (d_c * rs * g + bt).astype(o_ref.dtype)


def kernel(x, w1t, b1, w2t, b2, gamma, beta):
    seq, batch, d = x.shape
    f = w1t.shape[1]
    dtype = x.dtype

    xt = jnp.transpose(x, (0, 2, 1))
    w1 = w1t.T
    w2 = w2t.T
    b1r = b1.reshape(1, f)
    b2r = b2.reshape(1, d)
    gr = gamma.reshape(1, d)
    btr = beta.reshape(1, d)

    s_blk = min(32, seq)
    grid = (pl.cdiv(seq, s_blk),)

    n = seq * batch
    flops = 2 * n * d * f * 2 + 8 * n * d
    bytes_accessed = 4 * (2 * n * d + 2 * d * f + f + 3 * d)
    cost = pl.CostEstimate(flops=int(flops), transcendentals=int(n),
                           bytes_accessed=int(bytes_accessed))

    out_t = pl.pallas_call(
        functools.partial(_ffn_body, inv_d=1.0 / d),
        out_shape=jax.ShapeDtypeStruct((seq, d, batch), dtype),
        grid_spec=pltpu.PrefetchScalarGridSpec(
            num_scalar_prefetch=0,
            grid=grid,
            in_specs=[
                pl.BlockSpec((s_blk, d, batch), lambda i: (i, 0, 0)),
                pl.BlockSpec((f, d), lambda i: (0, 0)),
                pl.BlockSpec((1, f), lambda i: (0, 0)),
                pl.BlockSpec((d, f), lambda i: (0, 0)),
                pl.BlockSpec((1, d), lambda i: (0, 0)),
                pl.BlockSpec((1, d), lambda i: (0, 0)),
                pl.BlockSpec((1, d), lambda i: (0, 0)),
            ],
            out_specs=pl.BlockSpec((s_blk, d, batch), lambda i: (i, 0, 0)),
        ),
        compiler_params=pltpu.CompilerParams(
            dimension_semantics=("parallel",)),
        cost_estimate=cost,
    )(xt, w1, b1r, w2, b2r, gr, btr)

    return jnp.transpose(out_t, (0, 2, 1))
